# speedup vs baseline: 2.2912x; 2.2912x over previous
import math

import jax
import jax.numpy as jnp
from jax import lax
from jax.experimental import pallas as pl
from jax.experimental.pallas import tpu as pltpu
from jax.experimental.pallas import tpu_sc as plsc

VOCAB = 100000
D_MODEL = 128
BATCH = 4096
SEQ = 50

NC = 2
NS = 16
NW = NC * NS
L = 16

B_TOTAL = BATCH * SEQ
B_PER_W = B_TOTAL // NW
CHUNK = 128
N_CHUNKS = B_PER_W // CHUNK
VECS_PER_ROW = D_MODEL // L

SCALE = math.sqrt(float(D_MODEL))


def _emb_body(x_hbm, lut_hbm, out_hbm, idx_v, rows_v, sem):
    wid = lax.axis_index("s") * NC + lax.axis_index("c")
    base = wid * B_PER_W

    def chunk_body(c, carry):
        off = base + c * CHUNK
        pltpu.sync_copy(x_hbm.at[pl.ds(off, CHUNK)], idx_v)
        pltpu.async_copy(lut_hbm.at[idx_v], rows_v, sem).wait()

        def row_body(r, carry2):
            for j in range(VECS_PER_ROW):
                rows_v[r, pl.ds(j * L, L)] = rows_v[r, pl.ds(j * L, L)] * SCALE
            return carry2

        lax.fori_loop(0, CHUNK, row_body, 0)
        pltpu.sync_copy(rows_v, out_hbm.at[pl.ds(off, CHUNK)])
        return carry

    lax.fori_loop(0, N_CHUNKS, chunk_body, 0)


@jax.jit
def _emb(x_flat, lut):
    mesh = plsc.VectorSubcoreMesh(core_axis_name="c", subcore_axis_name="s")
    run = pl.kernel(
        _emb_body,
        out_type=jax.ShapeDtypeStruct((B_TOTAL, D_MODEL), jnp.float32),
        mesh=mesh,
        scratch_types=[
            pltpu.VMEM((CHUNK,), jnp.int32),
            pltpu.VMEM((CHUNK, D_MODEL), jnp.float32),
            pltpu.SemaphoreType.DMA,
        ],
    )
    return run(x_flat, lut)


def kernel(x, lut):
    out = _emb(x.reshape(B_TOTAL), lut)
    return out.reshape(BATCH, SEQ, D_MODEL)


# speedup vs baseline: 2.9498x; 1.2874x over previous
import math

import jax
import jax.numpy as jnp
from jax import lax
from jax.experimental import pallas as pl
from jax.experimental.pallas import tpu as pltpu
from jax.experimental.pallas import tpu_sc as plsc

VOCAB = 100000
D_MODEL = 128
BATCH = 4096
SEQ = 50

NC = 2
NS = 16
NW = NC * NS
L = 16

B_TOTAL = BATCH * SEQ
B_PER_W = B_TOTAL // NW
CHUNK = 128
N_CHUNKS = B_PER_W // CHUNK
NBUF = 2
N_GROUPS = N_CHUNKS // NBUF
VECS_PER_ROW = D_MODEL // L

SCALE = math.sqrt(float(D_MODEL))


def _emb_body(x_hbm, lut_hbm, out_hbm, idx_v, in_v, out_v, gsem, ssem):
    wid = lax.axis_index("s") * NC + lax.axis_index("c")
    base = wid * B_PER_W

    pltpu.sync_copy(x_hbm.at[pl.ds(base, B_PER_W)], idx_v)

    def fire_gather(c, b):
        pltpu.async_copy(
            lut_hbm.at[idx_v.at[pl.ds(c * CHUNK, CHUNK)]], in_v.at[b], gsem.at[b]
        )

    def wait_gather(c, b):
        pltpu.make_async_copy(
            lut_hbm.at[idx_v.at[pl.ds(c * CHUNK, CHUNK)]], in_v.at[b], gsem.at[b]
        ).wait()

    def fire_store(c, b):
        pltpu.async_copy(
            out_v.at[b], out_hbm.at[pl.ds(base + c * CHUNK, CHUNK)], ssem.at[b]
        )

    def wait_store(c, b):
        pltpu.make_async_copy(
            out_v.at[b], out_hbm.at[pl.ds(base + c * CHUNK, CHUNK)], ssem.at[b]
        ).wait()

    for b in range(NBUF):
        fire_gather(b, b)

    def group_body(g, carry):
        for b in range(NBUF):
            c = g * NBUF + b
            wait_gather(c, b)

            @pl.when(g > 0)
            def _():
                wait_store(c - NBUF, b)

            def row_body(r, carry2):
                for j in range(VECS_PER_ROW):
                    out_v[b, r, pl.ds(j * L, L)] = (
                        in_v[b, r, pl.ds(j * L, L)] * SCALE
                    )
                return carry2

            lax.fori_loop(0, CHUNK, row_body, 0)
            fire_store(c, b)

            @pl.when(g < N_GROUPS - 1)
            def _():
                fire_gather(c + NBUF, b)

        return carry

    lax.fori_loop(0, N_GROUPS, group_body, 0)

    for b in range(NBUF):
        wait_store(N_CHUNKS - NBUF + b, b)


@jax.jit
def _emb(x_flat, lut):
    mesh = plsc.VectorSubcoreMesh(core_axis_name="c", subcore_axis_name="s")
    run = pl.kernel(
        _emb_body,
        out_type=jax.ShapeDtypeStruct((B_TOTAL, D_MODEL), jnp.float32),
        mesh=mesh,
        scratch_types=[
            pltpu.VMEM((B_PER_W,), jnp.int32),
            pltpu.VMEM((NBUF, CHUNK, D_MODEL), jnp.float32),
            pltpu.VMEM((NBUF, CHUNK, D_MODEL), jnp.float32),
            pltpu.SemaphoreType.DMA((NBUF,)),
            pltpu.SemaphoreType.DMA((NBUF,)),
        ],
    )
    return run(x_flat, lut)


def kernel(x, lut):
    out = _emb(x.reshape(B_TOTAL), lut)
    return out.reshape(BATCH, SEQ, D_MODEL)


# speedup vs baseline: 5.2144x; 1.7677x over previous
import math

import jax
import jax.numpy as jnp
from jax import lax
from jax.experimental import pallas as pl
from jax.experimental.pallas import tpu as pltpu
from jax.experimental.pallas import tpu_sc as plsc

VOCAB = 100000
D_MODEL = 128
BATCH = 4096
SEQ = 50

NC = 2
NS = 16
NW = NC * NS
L = 16

B_TOTAL = BATCH * SEQ
B_PER_W = B_TOTAL // NW
BATCH_PER_W = BATCH // NW

CB = 4
CHUNK = CB * SEQ
GATHER_SPLITS = ((0, 128), (128, 72))
N_CHUNKS = BATCH_PER_W // CB
NBUF = 2
N_GROUPS = N_CHUNKS // NBUF
VECS_PER_ROW = D_MODEL // L

SCALE = math.sqrt(float(D_MODEL))


def _emb_body(x_hbm, lut_hbm, out_hbm, idx_v, in_v, out_v, gsem, ssem):
    wid = lax.axis_index("s") * NC + lax.axis_index("c")
    base = wid * B_PER_W
    bbase = wid * BATCH_PER_W

    pltpu.sync_copy(x_hbm.at[pl.ds(base, B_PER_W)], idx_v)

    def fire_gathers(c, b):
        for off, n in GATHER_SPLITS:
            pltpu.async_copy(
                lut_hbm.at[idx_v.at[pl.ds(c * CHUNK + off, n)]],
                in_v.at[b].at[pl.ds(off, n)],
                gsem.at[b],
            )

    def wait_gathers(c, b):
        for off, n in GATHER_SPLITS:
            pltpu.make_async_copy(
                lut_hbm.at[idx_v.at[pl.ds(c * CHUNK + off, n)]],
                in_v.at[b].at[pl.ds(off, n)],
                gsem.at[b],
            ).wait()

    def fire_store(c, b):
        pltpu.async_copy(
            out_v.at[b], out_hbm.at[pl.ds(bbase + c * CB, CB)], ssem.at[b]
        )

    def wait_store(c, b):
        pltpu.make_async_copy(
            out_v.at[b], out_hbm.at[pl.ds(bbase + c * CB, CB)], ssem.at[b]
        ).wait()

    for b in range(NBUF):
        fire_gathers(b, b)

    def group_body(g, carry):
        for b in range(NBUF):
            c = g * NBUF + b
            wait_gathers(c, b)

            @pl.when(g > 0)
            def _():
                wait_store(c - NBUF, b)

            for bi in range(CB):
                def row_body(s, carry2):
                    for j in range(VECS_PER_ROW):
                        out_v[b, bi, s, pl.ds(j * L, L)] = (
                            in_v[b, bi * SEQ + s, pl.ds(j * L, L)] * SCALE
                        )
                    return carry2

                lax.fori_loop(0, SEQ, row_body, 0)

            fire_store(c, b)

            @pl.when(g < N_GROUPS - 1)
            def _():
                fire_gathers(c + NBUF, b)

        return carry

    lax.fori_loop(0, N_GROUPS, group_body, 0)

    for b in range(NBUF):
        wait_store(N_CHUNKS - NBUF + b, b)


@jax.jit
def _emb(x_flat, lut):
    mesh = plsc.VectorSubcoreMesh(core_axis_name="c", subcore_axis_name="s")
    run = pl.kernel(
        _emb_body,
        out_type=jax.ShapeDtypeStruct((BATCH, SEQ, D_MODEL), jnp.float32),
        mesh=mesh,
        scratch_types=[
            pltpu.VMEM((B_PER_W,), jnp.int32),
            pltpu.VMEM((NBUF, CHUNK, D_MODEL), jnp.float32),
            pltpu.VMEM((NBUF, CB, SEQ, D_MODEL), jnp.float32),
            pltpu.SemaphoreType.DMA((NBUF,)),
            pltpu.SemaphoreType.DMA((NBUF,)),
        ],
    )
    return run(x_flat, lut)


def kernel(x, lut):
    return _emb(x.reshape(B_TOTAL), lut)
